# baseline (device time: 189513 ns/iter reference)
import jax
import jax.numpy as jnp
from jax import lax
from jax.experimental import pallas as pl
from jax.experimental.pallas import tpu as pltpu

N_DEV = 16
M = 4096
N = 2048
CHUNK = M // N_DEV
HALF = N // 2
QTR = HALF // 2

F8 = jnp.float8_e4m3fn

_NXT = [4, 0, 6, 2, 8, 1, 10, 3, 12, 5, 14, 7, 15, 9, 13, 11]
_PRV = [1, 5, 3, 7, 0, 9, 2, 11, 4, 13, 6, 15, 8, 14, 10, 12]
_ORD = [0, 15, 8, 7, 1, 14, 9, 6, 2, 13, 10, 5, 3, 12, 11, 4]

NB = 10


def kernel(x, w_mat):
    k_per = x.shape[1]
    assert x.shape == (M, k_per), x.shape
    assert w_mat.shape == (k_per, N), w_mat.shape

    def body(x_ref, w_ref, meta_ref, out_ref, xb_ref, wb_ref,
             fin_r, fin_l, comm_rb, comm_lb, comm_re, comm_le,
             send_r, recv_r, send_l, recv_l,
             esend_r, erecv_r, esend_l, erecv_l,
             q_r, q_l, qsend_r, qrecv_r, qsend_l, qrecv_l,
             amax_buf, asend, arecv):
        my = lax.axis_index("i")
        right = meta_ref[0]
        left = meta_ref[1]
        pos = meta_ref[2]

        barrier_sem = pltpu.get_barrier_semaphore()
        for nbr in (left, right):
            pl.semaphore_signal(
                barrier_sem, inc=1,
                device_id=(nbr,), device_id_type=pl.DeviceIdType.MESH,
            )
        pl.semaphore_wait(barrier_sem, 2)

        xb_ref[...] = x_ref[...].astype(jnp.bfloat16)
        wb_ref[...] = w_ref[...].astype(jnp.bfloat16)

        def partial_half(c, lo):
            xr = xb_ref[pl.ds(c * CHUNK, CHUNK), :]
            return jnp.dot(xr, wb_ref[:, lo:lo + HALF],
                           preferred_element_type=jnp.float32)

        R512 = jnp.float32(512.0)
        INV512 = jnp.float32(1.0 / 512.0)

        def rs_descs(s, dir_, sub):
            hib = comm_rb if dir_ == 0 else comm_lb
            ssem = send_r if dir_ == 0 else send_l
            rsem = recv_r if dir_ == 0 else recv_l
            tgt = right if dir_ == 0 else left
            lo = sub * QTR
            ds = [pltpu.make_async_remote_copy(
                src_ref=hib.at[s, :, lo:lo + QTR],
                dst_ref=hib.at[s + 1, :, lo:lo + QTR],
                send_sem=ssem.at[s, sub], recv_sem=rsem.at[s, sub],
                device_id=(tgt,), device_id_type=pl.DeviceIdType.MESH,
            )]
            if s >= NB:
                lob = comm_re if dir_ == 0 else comm_le
                esem = esend_r if dir_ == 0 else esend_l
                resem = erecv_r if dir_ == 0 else erecv_l
                ds.append(pltpu.make_async_remote_copy(
                    src_ref=lob.at[s, :, lo:lo + QTR],
                    dst_ref=lob.at[s + 1, :, lo:lo + QTR],
                    send_sem=esem.at[s, sub], recv_sem=resem.at[s, sub],
                    device_id=(tgt,), device_id_type=pl.DeviceIdType.MESH,
                ))
            return ds

        comm_rb[0] = partial_half(pos, 0).astype(jnp.bfloat16)
        comm_lb[0] = partial_half(pos, HALF).astype(jnp.bfloat16)
        pend_rs = []
        for dir_, sub in ((0, 0), (1, 0), (0, 1), (1, 1)):
            ds = rs_descs(0, dir_, sub)
            for d in ds:
                d.start()
            pend_rs.append((ds, dir_, sub))

        for s in range(N_DEV - 1):
            slot = s + 1
            dual = s >= NB
            dual_next = (s + 1) >= NB
            c_r = (pos + N_DEV - 1 - s) % N_DEV
            c_l = (pos + s + 1) % N_DEV
            p = (partial_half(c_r, 0), partial_half(c_l, HALF))
            nxt = []
            for ds, dir_, sub in pend_rs:
                for d in ds:
                    d.wait()
                hib = comm_rb if dir_ == 0 else comm_lb
                lo = sub * QTR
                got = hib[slot, :, lo:lo + QTR].astype(jnp.float32)
                if dual:
                    lob = comm_re if dir_ == 0 else comm_le
                    got = got + lob[slot, :, lo:lo + QTR].astype(
                        jnp.float32) * INV512
                acc = got + p[dir_][:, lo:lo + QTR]
                if s == N_DEV - 2:
                    fin = fin_r if dir_ == 0 else fin_l
                    fin[:, lo:lo + QTR] = acc
                else:
                    hi = acc.astype(jnp.bfloat16)
                    hib[slot, :, lo:lo + QTR] = hi
                    if dual_next:
                        lob = comm_re if dir_ == 0 else comm_le
                        lob[slot, :, lo:lo + QTR] = (
                            (acc - hi.astype(jnp.float32)) * R512).astype(F8)
                    nds = rs_descs(s + 1, dir_, sub)
                    for nd in nds:
                        nd.start()
                    nxt.append((nds, dir_, sub))
            pend_rs = nxt

        own_r = (pos + 1) % N_DEV
        own_l = (pos + N_DEV - 1) % N_DEV

        local_amax = jnp.maximum(jnp.max(jnp.abs(fin_r[...])),
                                 jnp.max(jnp.abs(fin_l[...])))
        amax_buf[my] = jnp.full((8, 128), local_amax, jnp.float32)
        sends = []
        for o in range(1, N_DEV):
            tgt = (my + o) % N_DEV
            rd = pltpu.make_async_remote_copy(
                src_ref=amax_buf.at[my], dst_ref=amax_buf.at[my],
                send_sem=asend.at[o], recv_sem=arecv.at[my],
                device_id=(tgt,), device_id_type=pl.DeviceIdType.MESH,
            )
            rd.start()
            sends.append(rd)
        for o in range(1, N_DEV):
            src = (my + o) % N_DEV
            rd = pltpu.make_async_remote_copy(
                src_ref=amax_buf.at[src], dst_ref=amax_buf.at[src],
                send_sem=asend.at[o], recv_sem=arecv.at[src],
                device_id=(src,), device_id_type=pl.DeviceIdType.MESH,
            )
            rd.wait_recv()
        for rd in sends:
            rd.wait_send()
        amax = jnp.max(amax_buf[:, 0, 0])
        scale = amax / 448.0
        inv_scale = 1.0 / scale

        def ag_sub(t, dir_, sub):
            buf = q_r if dir_ == 0 else q_l
            ssem = qsend_r if dir_ == 0 else qsend_l
            rsem = qrecv_r if dir_ == 0 else qrecv_l
            tgt = right if dir_ == 0 else left
            lo = sub * QTR
            return pltpu.make_async_remote_copy(
                src_ref=buf.at[t, :, lo:lo + QTR],
                dst_ref=buf.at[t + 1, :, lo:lo + QTR],
                send_sem=ssem.at[t, sub], recv_sem=rsem.at[t, sub],
                device_id=(tgt,), device_id_type=pl.DeviceIdType.MESH,
            )

        q_r[0] = (fin_r[...] * inv_scale).astype(F8)
        q_l[0] = (fin_l[...] * inv_scale).astype(F8)
        pend_ag = []
        for dir_, sub in ((0, 0), (1, 0), (0, 1), (1, 1)):
            d = ag_sub(0, dir_, sub)
            d.start()
            pend_ag.append((d, dir_, sub))
        out_ref[pl.ds(own_r * CHUNK, CHUNK), 0:HALF] = (
            q_r[0].astype(jnp.float32) * scale).astype(jnp.bfloat16)
        out_ref[pl.ds(own_l * CHUNK, CHUNK), HALF:N] = (
            q_l[0].astype(jnp.float32) * scale).astype(jnp.bfloat16)
        for t in range(N_DEV - 1):
            slot = t + 1
            c_r = (pos + N_DEV - t) % N_DEV
            c_l = (pos + t) % N_DEV
            nxt = []
            for d, dir_, sub in pend_ag:
                d.wait()
                if t < N_DEV - 2:
                    nd = ag_sub(t + 1, dir_, sub)
                    nd.start()
                    nxt.append((nd, dir_, sub))
                lo = sub * QTR
                if dir_ == 0:
                    out_ref[pl.ds(c_r * CHUNK, CHUNK), lo:lo + QTR] = (
                        q_r[slot, :, lo:lo + QTR].astype(jnp.float32)
                        * scale).astype(jnp.bfloat16)
                else:
                    out_ref[pl.ds(c_l * CHUNK, CHUNK),
                            HALF + lo:HALF + lo + QTR] = (
                        q_l[slot, :, lo:lo + QTR].astype(jnp.float32)
                        * scale).astype(jnp.bfloat16)
            pend_ag = nxt

    idx = lax.axis_index("i")
    meta = jnp.stack([
        jnp.asarray(_NXT, jnp.int32)[idx],
        jnp.asarray(_PRV, jnp.int32)[idx],
        jnp.asarray(_ORD, jnp.int32)[idx],
    ])

    return pl.pallas_call(
        body,
        out_shape=jax.ShapeDtypeStruct((M, N), jnp.bfloat16),
        in_specs=[
            pl.BlockSpec(memory_space=pltpu.VMEM),
            pl.BlockSpec(memory_space=pltpu.VMEM),
            pl.BlockSpec(memory_space=pltpu.SMEM),
        ],
        out_specs=pl.BlockSpec(memory_space=pltpu.VMEM),
        scratch_shapes=[
            pltpu.VMEM((M, k_per), jnp.bfloat16),
            pltpu.VMEM((k_per, N), jnp.bfloat16),
            pltpu.VMEM((CHUNK, HALF), jnp.float32),
            pltpu.VMEM((CHUNK, HALF), jnp.float32),
            pltpu.VMEM((N_DEV, CHUNK, HALF), jnp.bfloat16),
            pltpu.VMEM((N_DEV, CHUNK, HALF), jnp.bfloat16),
            pltpu.VMEM((N_DEV, CHUNK, HALF), F8),
            pltpu.VMEM((N_DEV, CHUNK, HALF), F8),
            pltpu.SemaphoreType.DMA((N_DEV - 1, 2)),
            pltpu.SemaphoreType.DMA((N_DEV - 1, 2)),
            pltpu.SemaphoreType.DMA((N_DEV - 1, 2)),
            pltpu.SemaphoreType.DMA((N_DEV - 1, 2)),
            pltpu.SemaphoreType.DMA((N_DEV - 1, 2)),
            pltpu.SemaphoreType.DMA((N_DEV - 1, 2)),
            pltpu.SemaphoreType.DMA((N_DEV - 1, 2)),
            pltpu.SemaphoreType.DMA((N_DEV - 1, 2)),
            pltpu.VMEM((N_DEV, CHUNK, HALF), F8),
            pltpu.VMEM((N_DEV, CHUNK, HALF), F8),
            pltpu.SemaphoreType.DMA((N_DEV - 1, 2)),
            pltpu.SemaphoreType.DMA((N_DEV - 1, 2)),
            pltpu.SemaphoreType.DMA((N_DEV - 1, 2)),
            pltpu.SemaphoreType.DMA((N_DEV - 1, 2)),
            pltpu.VMEM((N_DEV, 8, 128), jnp.float32),
            pltpu.SemaphoreType.DMA((N_DEV,)),
            pltpu.SemaphoreType.DMA((N_DEV,)),
        ],
        compiler_params=pltpu.CompilerParams(
            collective_id=7, vmem_limit_bytes=100 * 1024 * 1024
        ),
    )(x, w_mat, meta)


# device time: 183338 ns/iter; 1.0337x vs baseline; 1.0337x over previous
import jax
import jax.numpy as jnp
from jax import lax
from jax.experimental import pallas as pl
from jax.experimental.pallas import tpu as pltpu

N_DEV = 16
M = 4096
N = 2048
CHUNK = M // N_DEV
HALF = N // 2
QTR = HALF // 2

F8 = jnp.float8_e4m3fn

_NXT = [4, 0, 6, 2, 8, 1, 10, 3, 12, 5, 14, 7, 15, 9, 13, 11]
_PRV = [1, 5, 3, 7, 0, 9, 2, 11, 4, 13, 6, 15, 8, 14, 10, 12]
_ORD = [0, 15, 8, 7, 1, 14, 9, 6, 2, 13, 10, 5, 3, 12, 11, 4]

NB = 10


def kernel(x, w_mat):
    k_per = x.shape[1]
    assert x.shape == (M, k_per), x.shape
    assert w_mat.shape == (k_per, N), w_mat.shape

    def body(x_ref, w_ref, meta_ref, out_ref, xb_ref, wb_ref,
             fin_r, fin_l, comm_rb, comm_lb, comm_re, comm_le,
             send_r, recv_r, send_l, recv_l,
             esend_r, erecv_r, esend_l, erecv_l,
             q_r, q_l, qsend_r, qrecv_r, qsend_l, qrecv_l,
             amax_buf, asend, arecv):
        my = lax.axis_index("i")
        right = meta_ref[0, my]
        left = meta_ref[1, my]
        pos = meta_ref[2, my]

        barrier_sem = pltpu.get_barrier_semaphore()
        for nbr in (left, right):
            pl.semaphore_signal(
                barrier_sem, inc=1,
                device_id=(nbr,), device_id_type=pl.DeviceIdType.MESH,
            )
        pl.semaphore_wait(barrier_sem, 2)

        xb_ref[...] = x_ref[...].astype(jnp.bfloat16)
        wb_ref[...] = w_ref[...].astype(jnp.bfloat16)

        def partial_half(c, lo):
            xr = xb_ref[pl.ds(c * CHUNK, CHUNK), :]
            return jnp.dot(xr, wb_ref[:, lo:lo + HALF],
                           preferred_element_type=jnp.float32)

        R512 = jnp.float32(512.0)
        INV512 = jnp.float32(1.0 / 512.0)

        def rs_descs(s, dir_, sub):
            hib = comm_rb if dir_ == 0 else comm_lb
            ssem = send_r if dir_ == 0 else send_l
            rsem = recv_r if dir_ == 0 else recv_l
            tgt = right if dir_ == 0 else left
            lo = sub * QTR
            ds = [pltpu.make_async_remote_copy(
                src_ref=hib.at[s, :, lo:lo + QTR],
                dst_ref=hib.at[s + 1, :, lo:lo + QTR],
                send_sem=ssem.at[s, sub], recv_sem=rsem.at[s, sub],
                device_id=(tgt,), device_id_type=pl.DeviceIdType.MESH,
            )]
            if s >= NB:
                lob = comm_re if dir_ == 0 else comm_le
                esem = esend_r if dir_ == 0 else esend_l
                resem = erecv_r if dir_ == 0 else erecv_l
                ds.append(pltpu.make_async_remote_copy(
                    src_ref=lob.at[s, :, lo:lo + QTR],
                    dst_ref=lob.at[s + 1, :, lo:lo + QTR],
                    send_sem=esem.at[s, sub], recv_sem=resem.at[s, sub],
                    device_id=(tgt,), device_id_type=pl.DeviceIdType.MESH,
                ))
            return ds

        comm_rb[0] = partial_half(pos, 0).astype(jnp.bfloat16)
        comm_lb[0] = partial_half(pos, HALF).astype(jnp.bfloat16)
        pend_rs = []
        for dir_, sub in ((0, 0), (1, 0), (0, 1), (1, 1)):
            ds = rs_descs(0, dir_, sub)
            for d in ds:
                d.start()
            pend_rs.append((ds, dir_, sub))

        for s in range(N_DEV - 1):
            slot = s + 1
            dual = s >= NB
            dual_next = (s + 1) >= NB
            c_r = (pos + N_DEV - 1 - s) % N_DEV
            c_l = (pos + s + 1) % N_DEV
            p = (partial_half(c_r, 0), partial_half(c_l, HALF))
            nxt = []
            for ds, dir_, sub in pend_rs:
                for d in ds:
                    d.wait()
                hib = comm_rb if dir_ == 0 else comm_lb
                lo = sub * QTR
                got = hib[slot, :, lo:lo + QTR].astype(jnp.float32)
                if dual:
                    lob = comm_re if dir_ == 0 else comm_le
                    got = got + lob[slot, :, lo:lo + QTR].astype(
                        jnp.float32) * INV512
                acc = got + p[dir_][:, lo:lo + QTR]
                if s == N_DEV - 2:
                    fin = fin_r if dir_ == 0 else fin_l
                    fin[:, lo:lo + QTR] = acc
                else:
                    hi = acc.astype(jnp.bfloat16)
                    hib[slot, :, lo:lo + QTR] = hi
                    if dual_next:
                        lob = comm_re if dir_ == 0 else comm_le
                        lob[slot, :, lo:lo + QTR] = (
                            (acc - hi.astype(jnp.float32)) * R512).astype(F8)
                    nds = rs_descs(s + 1, dir_, sub)
                    for nd in nds:
                        nd.start()
                    nxt.append((nds, dir_, sub))
            pend_rs = nxt

        own_r = (pos + 1) % N_DEV
        own_l = (pos + N_DEV - 1) % N_DEV

        local_amax = jnp.maximum(jnp.max(jnp.abs(fin_r[...])),
                                 jnp.max(jnp.abs(fin_l[...])))
        amax_buf[my] = jnp.full((8, 128), local_amax, jnp.float32)
        sends = []
        for o in range(1, N_DEV):
            tgt = (my + o) % N_DEV
            rd = pltpu.make_async_remote_copy(
                src_ref=amax_buf.at[my], dst_ref=amax_buf.at[my],
                send_sem=asend.at[o], recv_sem=arecv.at[my],
                device_id=(tgt,), device_id_type=pl.DeviceIdType.MESH,
            )
            rd.start()
            sends.append(rd)
        for o in range(1, N_DEV):
            src = (my + o) % N_DEV
            rd = pltpu.make_async_remote_copy(
                src_ref=amax_buf.at[src], dst_ref=amax_buf.at[src],
                send_sem=asend.at[o], recv_sem=arecv.at[src],
                device_id=(src,), device_id_type=pl.DeviceIdType.MESH,
            )
            rd.wait_recv()
        for rd in sends:
            rd.wait_send()
        amax = jnp.max(amax_buf[:, 0, 0])
        scale = amax / 448.0
        inv_scale = 1.0 / scale

        def ag_sub(t, dir_, sub):
            buf = q_r if dir_ == 0 else q_l
            ssem = qsend_r if dir_ == 0 else qsend_l
            rsem = qrecv_r if dir_ == 0 else qrecv_l
            tgt = right if dir_ == 0 else left
            lo = sub * QTR
            return pltpu.make_async_remote_copy(
                src_ref=buf.at[t, :, lo:lo + QTR],
                dst_ref=buf.at[t + 1, :, lo:lo + QTR],
                send_sem=ssem.at[t, sub], recv_sem=rsem.at[t, sub],
                device_id=(tgt,), device_id_type=pl.DeviceIdType.MESH,
            )

        q_r[0] = (fin_r[...] * inv_scale).astype(F8)
        q_l[0] = (fin_l[...] * inv_scale).astype(F8)
        pend_ag = []
        for dir_, sub in ((0, 0), (1, 0), (0, 1), (1, 1)):
            d = ag_sub(0, dir_, sub)
            d.start()
            pend_ag.append((d, dir_, sub))
        out_ref[pl.ds(own_r * CHUNK, CHUNK), 0:HALF] = (
            q_r[0].astype(jnp.float32) * scale).astype(jnp.bfloat16)
        out_ref[pl.ds(own_l * CHUNK, CHUNK), HALF:N] = (
            q_l[0].astype(jnp.float32) * scale).astype(jnp.bfloat16)
        for t in range(N_DEV - 1):
            slot = t + 1
            c_r = (pos + N_DEV - t) % N_DEV
            c_l = (pos + t) % N_DEV
            nxt = []
            for d, dir_, sub in pend_ag:
                d.wait()
                if t < N_DEV - 2:
                    nd = ag_sub(t + 1, dir_, sub)
                    nd.start()
                    nxt.append((nd, dir_, sub))
                lo = sub * QTR
                if dir_ == 0:
                    out_ref[pl.ds(c_r * CHUNK, CHUNK), lo:lo + QTR] = (
                        q_r[slot, :, lo:lo + QTR].astype(jnp.float32)
                        * scale).astype(jnp.bfloat16)
                else:
                    out_ref[pl.ds(c_l * CHUNK, CHUNK),
                            HALF + lo:HALF + lo + QTR] = (
                        q_l[slot, :, lo:lo + QTR].astype(jnp.float32)
                        * scale).astype(jnp.bfloat16)
            pend_ag = nxt

    meta = jnp.asarray([_NXT, _PRV, _ORD], jnp.int32)

    return pl.pallas_call(
        body,
        out_shape=jax.ShapeDtypeStruct((M, N), jnp.bfloat16),
        in_specs=[
            pl.BlockSpec(memory_space=pltpu.VMEM),
            pl.BlockSpec(memory_space=pltpu.VMEM),
            pl.BlockSpec(memory_space=pltpu.SMEM),
        ],
        out_specs=pl.BlockSpec(memory_space=pltpu.VMEM),
        scratch_shapes=[
            pltpu.VMEM((M, k_per), jnp.bfloat16),
            pltpu.VMEM((k_per, N), jnp.bfloat16),
            pltpu.VMEM((CHUNK, HALF), jnp.float32),
            pltpu.VMEM((CHUNK, HALF), jnp.float32),
            pltpu.VMEM((N_DEV, CHUNK, HALF), jnp.bfloat16),
            pltpu.VMEM((N_DEV, CHUNK, HALF), jnp.bfloat16),
            pltpu.VMEM((N_DEV, CHUNK, HALF), F8),
            pltpu.VMEM((N_DEV, CHUNK, HALF), F8),
            pltpu.SemaphoreType.DMA((N_DEV - 1, 2)),
            pltpu.SemaphoreType.DMA((N_DEV - 1, 2)),
            pltpu.SemaphoreType.DMA((N_DEV - 1, 2)),
            pltpu.SemaphoreType.DMA((N_DEV - 1, 2)),
            pltpu.SemaphoreType.DMA((N_DEV - 1, 2)),
            pltpu.SemaphoreType.DMA((N_DEV - 1, 2)),
            pltpu.SemaphoreType.DMA((N_DEV - 1, 2)),
            pltpu.SemaphoreType.DMA((N_DEV - 1, 2)),
            pltpu.VMEM((N_DEV, CHUNK, HALF), F8),
            pltpu.VMEM((N_DEV, CHUNK, HALF), F8),
            pltpu.SemaphoreType.DMA((N_DEV - 1, 2)),
            pltpu.SemaphoreType.DMA((N_DEV - 1, 2)),
            pltpu.SemaphoreType.DMA((N_DEV - 1, 2)),
            pltpu.SemaphoreType.DMA((N_DEV - 1, 2)),
            pltpu.VMEM((N_DEV, 8, 128), jnp.float32),
            pltpu.SemaphoreType.DMA((N_DEV,)),
            pltpu.SemaphoreType.DMA((N_DEV,)),
        ],
        compiler_params=pltpu.CompilerParams(
            collective_id=7, vmem_limit_bytes=100 * 1024 * 1024
        ),
    )(x, w_mat, meta)
